# baseline (speedup 1.0000x reference)
import math

import jax
import jax.numpy as jnp
from jax import lax
from jax.experimental import pallas as pl
from jax.experimental.pallas import tpu as pltpu
from jax.experimental.pallas import tpu_sc as plsc

VOCAB = 50368
HIDDEN = 768
MAX_POS = 8192
B, S = 4, 8192
EPS = 1e-12

L = 16
NJ = HIDDEN // L
NC, NS = 2, 16
NW = NC * NS
SCHUNK = S // NW
T = 32
NCHUNK = SCHUNK // T
SCALE = math.sqrt(HIDDEN)
INV_H = 1.0 / HIDDEN


def _rsqrt_vec(v):
    i = lax.bitcast_convert_type(v, jnp.int32)
    y = lax.bitcast_convert_type(jnp.int32(0x5F3759DF) - (i >> 1), jnp.float32)
    for _ in range(3):
        y = y * (1.5 - 0.5 * v * y * y)
    return y


def _lanesum(v):
    lanes = lax.iota(jnp.int32, L)
    for k in (8, 4, 2, 1):
        v = v + v.at[lanes ^ k].get(mode="promise_in_bounds")
    return v


def _body(ids_hbm, word_hbm, pos_hbm, tt_hbm, gam_hbm, bet_hbm, out_hbm,
          ids_v, pos_v, word_v, tt_v, gam_v, bet_v, sem):
    cid = lax.axis_index("c")
    sid = lax.axis_index("s")
    wid = sid * NC + cid
    s0 = wid * SCHUNK

    pltpu.sync_copy(ids_hbm.at[:, pl.ds(s0, SCHUNK)], ids_v)
    pltpu.sync_copy(tt_hbm.at[0], tt_v)
    pltpu.sync_copy(gam_hbm, gam_v)
    pltpu.sync_copy(bet_hbm, bet_v)

    def chunk_body(c, carry):
        pltpu.sync_copy(pos_hbm.at[pl.ds(s0 + c * T, T)], pos_v)

        def posadd(t, _):
            for j in range(NJ):
                sl = pl.ds(j * L, L)
                pos_v[t, sl] = pos_v[t, sl] + tt_v[sl]
            return _

        lax.fori_loop(0, T, posadd, None)

        def batch_body(b, _):
            idx = ids_v.at[b, pl.ds(c * T, T)]
            pltpu.async_copy(word_hbm.at[idx], word_v, sem).wait()

            def tok(t, __):
                acc_s = jnp.zeros((L,), jnp.float32)
                acc_q = jnp.zeros((L,), jnp.float32)
                for j in range(NJ):
                    sl = pl.ds(j * L, L)
                    x = word_v[t, sl] * SCALE + pos_v[t, sl]
                    word_v[t, sl] = x
                    acc_s = acc_s + x
                    acc_q = acc_q + x * x
                mv = _lanesum(acc_s) * INV_H
                var = _lanesum(acc_q) * INV_H - mv * mv
                var = jnp.maximum(var, 0.0)
                r = _rsqrt_vec(var + EPS)
                for j in range(NJ):
                    sl = pl.ds(j * L, L)
                    word_v[t, sl] = ((word_v[t, sl] - mv) * r) * gam_v[sl] + bet_v[sl]
                return __

            lax.fori_loop(0, T, tok, None)
            pltpu.sync_copy(word_v, out_hbm.at[b, pl.ds(s0 + c * T, T), :])
            return _

        lax.fori_loop(0, B, batch_body, None)
        return carry

    lax.fori_loop(0, NCHUNK, chunk_body, None)


def kernel(input_ids, word_table, pos_table, tt_table, ln_gamma, ln_beta):
    mesh = plsc.VectorSubcoreMesh(core_axis_name="c", subcore_axis_name="s")
    k = pl.kernel(
        _body,
        mesh=mesh,
        out_type=jax.ShapeDtypeStruct((B, S, HIDDEN), jnp.float32),
        scratch_types=[
            pltpu.VMEM((B, SCHUNK), jnp.int32),
            pltpu.VMEM((T, HIDDEN), jnp.float32),
            pltpu.VMEM((T, HIDDEN), jnp.float32),
            pltpu.VMEM((HIDDEN,), jnp.float32),
            pltpu.VMEM((HIDDEN,), jnp.float32),
            pltpu.VMEM((HIDDEN,), jnp.float32),
            pltpu.SemaphoreType.DMA,
        ],
    )
    return k(input_ids, word_table, pos_table, tt_table, ln_gamma, ln_beta)


# speedup vs baseline: 2.1007x; 2.1007x over previous
import math

import jax
import jax.numpy as jnp
from jax import lax
from jax.experimental import pallas as pl
from jax.experimental.pallas import tpu as pltpu
from jax.experimental.pallas import tpu_sc as plsc

VOCAB = 50368
HIDDEN = 768
MAX_POS = 8192
B, S = 4, 8192
EPS = 1e-12

L = 16
NJ = HIDDEN // L
NC, NS = 2, 16
NW = NC * NS
SCHUNK = S // NW
T = 8
NCHUNK = SCHUNK // T
NBUF = 4
SCALE = math.sqrt(HIDDEN)
INV_H = 1.0 / HIDDEN


def _rsqrt_vec(v):
    i = lax.bitcast_convert_type(v, jnp.int32)
    y = lax.bitcast_convert_type(jnp.int32(0x5F3759DF) - (i >> 1), jnp.float32)
    for _ in range(3):
        y = y * (1.5 - 0.5 * v * y * y)
    return y


def _splat_i32(x):
    return jnp.full((L,), x, jnp.int32)


_BFLY = [8, 4, 2, 1]


def _lanesum(v, lanes):
    for k in _BFLY:
        v = v + v.at[lanes ^ k].get(mode="promise_in_bounds")
    return v


def _body(ids_hbm, word_hbm, pos_hbm, tt_hbm, gam_hbm, bet_hbm, out_hbm,
          ids_v, word_v, pos_v, m_v, r_v, tt_v, gam_v, bet_v,
          gsems, psems, osems):
    cid = lax.axis_index("c")
    sid = lax.axis_index("s")
    wid = sid * NC + cid
    s0 = wid * SCHUNK

    pltpu.sync_copy(ids_hbm.at[:, pl.ds(s0, SCHUNK)], ids_v)
    pltpu.sync_copy(tt_hbm.at[0], tt_v)
    pltpu.sync_copy(gam_hbm, gam_v)
    pltpu.sync_copy(bet_hbm, bet_v)

    def gather_copies(c, buf):
        copies = [
            pltpu.make_async_copy(
                word_hbm.at[ids_v.at[b, pl.ds(c * T, T)]],
                word_v.at[buf, b],
                gsems[buf],
            )
            for b in range(B)
        ]
        copies.append(
            pltpu.make_async_copy(
                pos_hbm.at[pl.ds(s0 + c * T, T)], pos_v.at[buf % 2], psems[buf % 2]
            )
        )
        return copies

    def out_copies(c, buf):
        return [
            pltpu.make_async_copy(
                word_v.at[buf, b],
                out_hbm.at[b, pl.ds(s0 + c * T, T), :],
                osems[buf],
            )
            for b in range(B)
        ]

    for cp in gather_copies(0, 0):
        cp.start()

    iota = lax.iota(jnp.int32, L)

    def step(c, buf):
        nbuf = (buf + 1) % NBUF
        @pl.when(c >= NBUF - 1)
        def _():
            for cp in out_copies(c - (NBUF - 1), nbuf):
                cp.wait()

        @pl.when(c + 1 < NCHUNK)
        def _():
            for cp in gather_copies(c + 1, nbuf):
                cp.start()

        for cp in gather_copies(c, buf):
            cp.wait()

        def pass_a(t, carry):
            s = [jnp.zeros((L,), jnp.float32) for _ in range(B)]
            q = [jnp.zeros((L,), jnp.float32) for _ in range(B)]
            for j in range(NJ):
                sl = pl.ds(j * L, L)
                p = pos_v[buf % 2, t, sl] + tt_v[sl]
                for b in range(B):
                    x = word_v[buf, b, t, sl] * SCALE + p
                    word_v[buf, b, t, sl] = x
                    s[b] = s[b] + x
                    q[b] = q[b] + x * x
            for b in range(B):
                mean = _lanesum(s[b], iota) * INV_H
                var = jnp.maximum(
                    _lanesum(q[b], iota) * INV_H - mean * mean, 0.0
                ) + EPS
                m_v[b, t, :] = mean
                r_v[b, t, :] = _rsqrt_vec(var)
            return carry

        lax.fori_loop(0, T, pass_a, 0, unroll=False)

        def pass_b(t, carry):
            msp = [m_v[b, t, :] for b in range(B)]
            rsp = [r_v[b, t, :] for b in range(B)]
            for j in range(NJ):
                sl = pl.ds(j * L, L)
                g = gam_v[sl]
                bt = bet_v[sl]
                for b in range(B):
                    x = word_v[buf, b, t, sl]
                    word_v[buf, b, t, sl] = ((x - msp[b]) * rsp[b]) * g + bt
            return carry

        lax.fori_loop(0, T, pass_b, 0, unroll=False)

        for cp in out_copies(c, buf):
            cp.start()

    def ring_body(p, carry):
        for k in range(NBUF):
            step(p * NBUF + k, k)
        return carry

    lax.fori_loop(0, NCHUNK // NBUF, ring_body, 0, unroll=False)

    for k in range(1, NBUF):
        c = NCHUNK - NBUF + k
        for cp in out_copies(c, c % NBUF):
            cp.wait()


def kernel(input_ids, word_table, pos_table, tt_table, ln_gamma, ln_beta):
    mesh = plsc.VectorSubcoreMesh(core_axis_name="c", subcore_axis_name="s")
    k = pl.kernel(
        _body,
        mesh=mesh,
        out_type=jax.ShapeDtypeStruct((B, S, HIDDEN), jnp.float32),
        scratch_types=[
            pltpu.VMEM((B, SCHUNK), jnp.int32),
            pltpu.VMEM((NBUF, B, T, HIDDEN), jnp.float32),
            pltpu.VMEM((2, T, HIDDEN), jnp.float32),
            pltpu.VMEM((B, T, L), jnp.float32),
            pltpu.VMEM((B, T, L), jnp.float32),
            pltpu.VMEM((HIDDEN,), jnp.float32),
            pltpu.VMEM((HIDDEN,), jnp.float32),
            pltpu.VMEM((HIDDEN,), jnp.float32),
            [pltpu.SemaphoreType.DMA] * NBUF,
            [pltpu.SemaphoreType.DMA] * 2,
            [pltpu.SemaphoreType.DMA] * NBUF,
        ],
    )
    return k(input_ids, word_table, pos_table, tt_table, ln_gamma, ln_beta)
